# baseline (speedup 1.0000x reference)
import functools

import jax
import jax.numpy as jnp
from jax.experimental import pallas as pl
from jax.experimental.pallas import tpu as pltpu

CODEBOOK_SIZE = 8192
CODEBOOK_DIM = 256
EPSILON = 1e-05

BM = 512
BK = 1024


def _body(nk, a_ref, es_ref, u_ref, o_ref, minval_ref, minidx_ref):
    k = pl.program_id(1)
    a = a_ref[...]
    es = es_ref[...]
    u = u_ref[...]
    embed = es / jnp.maximum(u, EPSILON)
    b2 = jnp.sum(embed * embed, axis=1)[None, :]
    a2 = jnp.sum(a * a, axis=1, keepdims=True)
    prod = jax.lax.dot_general(
        a, embed, (((1,), (1,)), ((), ())),
        preferred_element_type=jnp.float32)
    d2 = jnp.maximum(a2 + b2 - 2.0 * prod, 0.0)

    local_min = jnp.min(d2, axis=1, keepdims=True)
    ids = jax.lax.broadcasted_iota(jnp.int32, d2.shape, 1)
    local_idx = jnp.min(
        jnp.where(d2 == local_min, ids, jnp.int32(CODEBOOK_SIZE)),
        axis=1, keepdims=True) + k * BK

    @pl.when(k == 0)
    def _init():
        minval_ref[...] = local_min
        minidx_ref[...] = local_idx

    @pl.when(k > 0)
    def _update():
        better = local_min < minval_ref[...]
        minval_ref[...] = jnp.where(better, local_min, minval_ref[...])
        minidx_ref[...] = jnp.where(better, local_idx, minidx_ref[...])

    @pl.when(k == nk - 1)
    def _emit():
        o_ref[...] = minidx_ref[...]


def kernel(hidden_states, embed_sum, cluster_usage):
    shape = hidden_states.shape
    flat = hidden_states.reshape(-1, shape[-1]).astype(jnp.float32)
    m, d = flat.shape
    kk = embed_sum.shape[0]
    nm = m // BM
    nk = kk // BK
    usage = cluster_usage.reshape(kk, 1)

    out = pl.pallas_call(
        functools.partial(_body, nk),
        grid=(nm, nk),
        in_specs=[
            pl.BlockSpec((BM, d), lambda i, j: (i, 0)),
            pl.BlockSpec((BK, d), lambda i, j: (j, 0)),
            pl.BlockSpec((BK, 1), lambda i, j: (j, 0)),
        ],
        out_specs=pl.BlockSpec((BM, 1), lambda i, j: (i, 0)),
        out_shape=jax.ShapeDtypeStruct((m, 1), jnp.int32),
        scratch_shapes=[
            pltpu.VMEM((BM, 1), jnp.float32),
            pltpu.VMEM((BM, 1), jnp.int32),
        ],
    )(flat, embed_sum, usage)
    return out.reshape(shape[:-1])


# speedup vs baseline: 1.0129x; 1.0129x over previous
import functools

import jax
import jax.numpy as jnp
from jax.experimental import pallas as pl
from jax.experimental.pallas import tpu as pltpu

CODEBOOK_SIZE = 8192
CODEBOOK_DIM = 256
EPSILON = 1e-05

BM = 512
BK = 1024
DAUG = CODEBOOK_DIM + 8


def _body(nk, a_ref, es_ref, u_ref, o_ref, eaug_ref, minval_ref, minidx_ref):
    i = pl.program_id(0)
    j = pl.program_id(1)

    @pl.when(i == 0)
    def _prep():
        es = es_ref[pl.ds(j * BK, BK), :]
        u = u_ref[pl.ds(j * BK, BK), :]
        em = es / jnp.maximum(u, EPSILON)
        b2 = jnp.sum(em * em, axis=1, keepdims=True)
        aug = jnp.concatenate(
            [-2.0 * em, b2, jnp.zeros((BK, DAUG - CODEBOOK_DIM - 1), jnp.float32)],
            axis=1)
        eaug_ref[pl.ds(j * BK, BK), :] = aug

    a = a_ref[...]
    eaug = eaug_ref[pl.ds(j * BK, BK), :]
    score = jax.lax.dot_general(
        a, eaug, (((1,), (1,)), ((), ())),
        preferred_element_type=jnp.float32)

    local_min = jnp.min(score, axis=1, keepdims=True)
    ids = jax.lax.broadcasted_iota(jnp.int32, score.shape, 1)
    local_idx = jnp.min(
        jnp.where(score == local_min, ids, jnp.int32(CODEBOOK_SIZE)),
        axis=1, keepdims=True) + j * BK

    @pl.when(j == 0)
    def _init():
        minval_ref[...] = local_min
        minidx_ref[...] = local_idx

    @pl.when(j > 0)
    def _update():
        better = local_min < minval_ref[...]
        minval_ref[...] = jnp.where(better, local_min, minval_ref[...])
        minidx_ref[...] = jnp.where(better, local_idx, minidx_ref[...])

    @pl.when(j == nk - 1)
    def _emit():
        o_ref[...] = minidx_ref[...]


def kernel(hidden_states, embed_sum, cluster_usage):
    shape = hidden_states.shape
    flat = hidden_states.reshape(-1, shape[-1]).astype(jnp.float32)
    m, d = flat.shape
    kk = embed_sum.shape[0]
    nm = m // BM
    nk = kk // BK
    usage = cluster_usage.reshape(kk, 1)
    a_aug = jnp.concatenate(
        [flat, jnp.ones((m, 1), jnp.float32), jnp.zeros((m, DAUG - d - 1), jnp.float32)],
        axis=1)

    out = pl.pallas_call(
        functools.partial(_body, nk),
        grid=(nm, nk),
        in_specs=[
            pl.BlockSpec((BM, DAUG), lambda i, j: (i, 0)),
            pl.BlockSpec((kk, d), lambda i, j: (0, 0)),
            pl.BlockSpec((kk, 1), lambda i, j: (0, 0)),
        ],
        out_specs=pl.BlockSpec((BM, 1), lambda i, j: (i, 0)),
        out_shape=jax.ShapeDtypeStruct((m, 1), jnp.int32),
        scratch_shapes=[
            pltpu.VMEM((kk, DAUG), jnp.float32),
            pltpu.VMEM((BM, 1), jnp.float32),
            pltpu.VMEM((BM, 1), jnp.int32),
        ],
    )(a_aug, embed_sum, usage)
    return out.reshape(shape[:-1])


# speedup vs baseline: 1.1146x; 1.1005x over previous
import functools

import jax
import jax.numpy as jnp
from jax.experimental import pallas as pl
from jax.experimental.pallas import tpu as pltpu

CODEBOOK_SIZE = 8192
CODEBOOK_DIM = 256
EPSILON = 1e-05

BM = 512
BK = 1024


def _body(nk, a_ref, es_ref, u_ref, o_ref,
          em2_ref, b2_ref, a2_ref, minval_ref, minidx_ref):
    i = pl.program_id(0)
    j = pl.program_id(1)

    @pl.when(i == 0)
    def _prep_codebook():
        es = es_ref[pl.ds(j * BK, BK), :]
        u = u_ref[pl.ds(j * BK, BK), :]
        em = es / jnp.maximum(u, EPSILON)
        b2_ref[:, pl.ds(j * BK, BK)] = jnp.sum(em * em, axis=1)[None, :]
        em2_ref[pl.ds(j * BK, BK), :] = -2.0 * em

    a = a_ref[...]

    @pl.when(j == 0)
    def _prep_rows():
        a2_ref[...] = jnp.sum(a * a, axis=1, keepdims=True)

    em2 = em2_ref[pl.ds(j * BK, BK), :]
    prod2 = jax.lax.dot_general(
        a, em2, (((1,), (1,)), ((), ())),
        preferred_element_type=jnp.float32)
    b2 = b2_ref[:, pl.ds(j * BK, BK)]
    d2 = (a2_ref[...] + b2) + prod2

    local_min = jnp.min(d2, axis=1, keepdims=True)
    ids = jax.lax.broadcasted_iota(jnp.int32, d2.shape, 1)
    local_idx = jnp.min(
        jnp.where(d2 == local_min, ids, jnp.int32(CODEBOOK_SIZE)),
        axis=1, keepdims=True) + j * BK

    @pl.when(j == 0)
    def _init():
        minval_ref[...] = local_min
        minidx_ref[...] = local_idx

    @pl.when(j > 0)
    def _update():
        better = local_min < minval_ref[...]
        minval_ref[...] = jnp.where(better, local_min, minval_ref[...])
        minidx_ref[...] = jnp.where(better, local_idx, minidx_ref[...])

    @pl.when(j == nk - 1)
    def _emit():
        o_ref[...] = minidx_ref[...]


def kernel(hidden_states, embed_sum, cluster_usage):
    shape = hidden_states.shape
    flat = hidden_states.reshape(-1, shape[-1]).astype(jnp.float32)
    m, d = flat.shape
    kk = embed_sum.shape[0]
    nm = m // BM
    nk = kk // BK
    usage = cluster_usage.reshape(kk, 1)

    out = pl.pallas_call(
        functools.partial(_body, nk),
        grid=(nm, nk),
        in_specs=[
            pl.BlockSpec((BM, d), lambda i, j: (i, 0)),
            pl.BlockSpec((kk, d), lambda i, j: (0, 0)),
            pl.BlockSpec((kk, 1), lambda i, j: (0, 0)),
        ],
        out_specs=pl.BlockSpec((BM, 1), lambda i, j: (i, 0)),
        out_shape=jax.ShapeDtypeStruct((m, 1), jnp.int32),
        scratch_shapes=[
            pltpu.VMEM((kk, d), jnp.float32),
            pltpu.VMEM((1, kk), jnp.float32),
            pltpu.VMEM((BM, 1), jnp.float32),
            pltpu.VMEM((BM, 1), jnp.float32),
            pltpu.VMEM((BM, 1), jnp.int32),
        ],
    )(flat, embed_sum, usage)
    return out.reshape(shape[:-1])


# speedup vs baseline: 1.2760x; 1.1447x over previous
import functools

import jax
import jax.numpy as jnp
from jax.experimental import pallas as pl
from jax.experimental.pallas import tpu as pltpu

CODEBOOK_SIZE = 8192
CODEBOOK_DIM = 256
EPSILON = 1e-05

BM = 512
BK = 1024


def _body(nk, a_ref, es_ref, u_ref, o_ref,
          em2_ref, b2_ref, minval_ref, minidx_ref):
    i = pl.program_id(0)
    j = pl.program_id(1)

    @pl.when(i == 0)
    def _prep_codebook():
        es = es_ref[pl.ds(j * BK, BK), :]
        u = u_ref[pl.ds(j * BK, BK), :]
        em = es / jnp.maximum(u, EPSILON)
        b2_ref[:, pl.ds(j * BK, BK)] = jnp.sum(em * em, axis=1)[None, :]
        em2_ref[pl.ds(j * BK, BK), :] = -2.0 * em

    a = a_ref[...]

    em2 = em2_ref[pl.ds(j * BK, BK), :]
    prod2 = jax.lax.dot_general(
        a, em2, (((1,), (1,)), ((), ())),
        preferred_element_type=jnp.float32)
    b2 = b2_ref[:, pl.ds(j * BK, BK)]
    d2 = b2 + prod2

    local_min = jnp.min(d2, axis=1, keepdims=True)
    ids = jax.lax.broadcasted_iota(jnp.int32, d2.shape, 1)
    local_idx = jnp.min(
        jnp.where(d2 == local_min, ids, jnp.int32(CODEBOOK_SIZE)),
        axis=1, keepdims=True) + j * BK

    @pl.when(j == 0)
    def _init():
        minval_ref[...] = local_min
        minidx_ref[...] = local_idx

    @pl.when(j > 0)
    def _update():
        better = local_min < minval_ref[...]
        minval_ref[...] = jnp.where(better, local_min, minval_ref[...])
        minidx_ref[...] = jnp.where(better, local_idx, minidx_ref[...])

    @pl.when(j == nk - 1)
    def _emit():
        o_ref[...] = minidx_ref[...]


def kernel(hidden_states, embed_sum, cluster_usage):
    shape = hidden_states.shape
    flat = hidden_states.reshape(-1, shape[-1]).astype(jnp.float32)
    m, d = flat.shape
    kk = embed_sum.shape[0]
    nm = m // BM
    nk = kk // BK
    usage = cluster_usage.reshape(kk, 1)

    out = pl.pallas_call(
        functools.partial(_body, nk),
        grid=(nm, nk),
        in_specs=[
            pl.BlockSpec((BM, d), lambda i, j: (i, 0)),
            pl.BlockSpec((kk, d), lambda i, j: (0, 0)),
            pl.BlockSpec((kk, 1), lambda i, j: (0, 0)),
        ],
        out_specs=pl.BlockSpec((BM, 1), lambda i, j: (i, 0)),
        out_shape=jax.ShapeDtypeStruct((m, 1), jnp.int32),
        scratch_shapes=[
            pltpu.VMEM((kk, d), jnp.float32),
            pltpu.VMEM((1, kk), jnp.float32),
            pltpu.VMEM((BM, 1), jnp.float32),
            pltpu.VMEM((BM, 1), jnp.int32),
        ],
    )(flat, embed_sum, usage)
    return out.reshape(shape[:-1])


# speedup vs baseline: 1.3865x; 1.0866x over previous
import functools

import jax
import jax.numpy as jnp
from jax.experimental import pallas as pl
from jax.experimental.pallas import tpu as pltpu

CODEBOOK_SIZE = 8192
CODEBOOK_DIM = 256
EPSILON = 1e-05

BM = 512
BK = 1024


def _local_min_idx(d2, base):
    lmin = jnp.min(d2, axis=1, keepdims=True)
    ids = jax.lax.broadcasted_iota(jnp.int32, (1, BK), 1)
    lidx = jnp.min(
        jnp.where(d2 == lmin, ids, jnp.int32(CODEBOOK_SIZE)),
        axis=1, keepdims=True) + base
    return lmin, lidx


def _fold(lmin, lidx, minval_ref, minidx_ref):
    better = lmin < minval_ref[...]
    minval_ref[...] = jnp.where(better, lmin, minval_ref[...])
    minidx_ref[...] = jnp.where(better, lidx, minidx_ref[...])


def _body(nt, a_ref, es_ref, u_ref, o_ref,
          em2_ref, b2_ref, pa_ref, pb_ref, minval_ref, minidx_ref):
    i = pl.program_id(0)
    t = pl.program_id(1)
    j0 = 2 * t

    @pl.when(i == 0)
    def _prep_codebook():
        es = es_ref[pl.ds(j0 * BK, 2 * BK), :]
        u = u_ref[pl.ds(j0 * BK, 2 * BK), :]
        em = es / jnp.maximum(u, EPSILON)
        b2_ref[:, pl.ds(j0 * BK, 2 * BK)] = jnp.sum(em * em, axis=1)[None, :]
        em2_ref[pl.ds(j0 * BK, 2 * BK), :] = -2.0 * em

    a = a_ref[...]

    pa_ref[...] = jax.lax.dot_general(
        a, em2_ref[pl.ds(j0 * BK, BK), :], (((1,), (1,)), ((), ())),
        preferred_element_type=jnp.float32)

    jprev = jnp.maximum(j0 - 1, 0)
    d2p = b2_ref[:, pl.ds(jprev * BK, BK)] + pb_ref[...]
    lminp, lidxp = _local_min_idx(d2p, jprev * BK)

    @pl.when(t > 0)
    def _fold_prev():
        _fold(lminp, lidxp, minval_ref, minidx_ref)

    pb_ref[...] = jax.lax.dot_general(
        a, em2_ref[pl.ds((j0 + 1) * BK, BK), :], (((1,), (1,)), ((), ())),
        preferred_element_type=jnp.float32)

    d2a = b2_ref[:, pl.ds(j0 * BK, BK)] + pa_ref[...]
    lmina, lidxa = _local_min_idx(d2a, j0 * BK)

    @pl.when(t == 0)
    def _init():
        minval_ref[...] = lmina
        minidx_ref[...] = lidxa

    @pl.when(t > 0)
    def _fold_a():
        _fold(lmina, lidxa, minval_ref, minidx_ref)

    @pl.when(t == nt - 1)
    def _tail():
        d2b = b2_ref[:, pl.ds((j0 + 1) * BK, BK)] + pb_ref[...]
        lminb, lidxb = _local_min_idx(d2b, (j0 + 1) * BK)
        _fold(lminb, lidxb, minval_ref, minidx_ref)
        o_ref[...] = minidx_ref[...]


def kernel(hidden_states, embed_sum, cluster_usage):
    shape = hidden_states.shape
    flat = hidden_states.reshape(-1, shape[-1]).astype(jnp.float32)
    m, d = flat.shape
    kk = embed_sum.shape[0]
    nm = m // BM
    nt = kk // (2 * BK)
    usage = cluster_usage.reshape(kk, 1)

    out = pl.pallas_call(
        functools.partial(_body, nt),
        grid=(nm, nt),
        in_specs=[
            pl.BlockSpec((BM, d), lambda i, t: (i, 0)),
            pl.BlockSpec((kk, d), lambda i, t: (0, 0)),
            pl.BlockSpec((kk, 1), lambda i, t: (0, 0)),
        ],
        out_specs=pl.BlockSpec((BM, 1), lambda i, t: (i, 0)),
        out_shape=jax.ShapeDtypeStruct((m, 1), jnp.int32),
        scratch_shapes=[
            pltpu.VMEM((kk, d), jnp.float32),
            pltpu.VMEM((1, kk), jnp.float32),
            pltpu.VMEM((BM, BK), jnp.float32),
            pltpu.VMEM((BM, BK), jnp.float32),
            pltpu.VMEM((BM, 1), jnp.float32),
            pltpu.VMEM((BM, 1), jnp.int32),
        ],
    )(flat, embed_sum, usage)
    return out.reshape(shape[:-1])
